# baseline (device time: 45481 ns/iter reference)
import jax
import jax.numpy as jnp
from jax import lax
from jax.experimental import pallas as pl
from jax.experimental.pallas import tpu as pltpu

N_DEV = 8
M = 1024
D = 1024

SIZES = (192, 160, 192, 160, 160, 160)
BASES = (0, 192, 352, 544, 704, 864)
ORDERS = (
    ("x", "y", "z"), ("x", "z", "y"),
    ("y", "z", "x"), ("y", "x", "z"),
    ("z", "x", "y"), ("z", "y", "x"),
)
RB_OFFS = tuple((0, s // 2, 3 * s // 4) for s in SIZES)
RES_OFFS = (0, 48, 88, 136, 176, 216)
ORDER = (1, 3, 4, 5, 0, 2)


def _partner_and_bit(p, axis):
    q = lax.rem(p, 4)
    zc = lax.div(p, 4)
    if axis == "x":
        return zc * 4 + jnp.bitwise_xor(q, 1), jnp.bitwise_and(
            jnp.bitwise_xor(q, lax.div(q, 2)), 1
        )
    if axis == "y":
        return zc * 4 + (3 - q), lax.div(q, 2)
    return jnp.bitwise_xor(p, 4), zc


def kernel(partial, resid, gamma):
    gamma2 = gamma.reshape(1, D)

    def body(x_ref, resid_ref, gamma_ref, out_ref,
             acc, ov, rb0, rb1, rb2, rb3, rb4, rb5, res_loc,
             send_sems, recv_sems, res_sems, xcp_sems, ocp_sems):
        p = lax.axis_index("i")
        rbufs = (rb0, rb1, rb2, rb3, rb4, rb5)

        pb = [[_partner_and_bit(p, ORDERS[c][j]) for j in range(3)]
              for c in range(6)]

        barrier_sem = pltpu.get_barrier_semaphore()
        for axis in ("x", "y", "z"):
            pl.semaphore_signal(
                barrier_sem, inc=1,
                device_id=(_partner_and_bit(p, axis)[0],),
                device_id_type=pl.DeviceIdType.MESH,
            )
        pl.semaphore_wait(barrier_sem, 3)

        x_copies = []
        res_copies = []
        for c in range(6):
            s = SIZES[c]
            b0, b1 = pb[c][0][1], pb[c][1][1]
            keep0 = BASES[c] + b0 * (s >> 1)
            cp = pltpu.make_async_copy(
                x_ref.at[0, pl.ds(keep0, s >> 1), :],
                acc.at[pl.ds(keep0, s >> 1), :],
                xcp_sems.at[c],
            )
            cp.start()
            x_copies.append(cp)
            off_fin = keep0 + b1 * (s >> 2)
            cp = pltpu.make_async_copy(
                resid_ref.at[pl.ds(off_fin, s >> 2), :],
                res_loc.at[pl.ds(RES_OFFS[c], s >> 2), :],
                res_sems.at[c],
            )
            cp.start()
            res_copies.append(cp)

        def make_rdma(c, ph, offs):
            s = SIZES[c]
            partner = pb[c][(0, 1, 2, 1, 0)[ph]][0]
            if ph == 0:
                half = s >> 1
                b = pb[c][0][1]
                src = x_ref.at[0, pl.ds(offs[c] + (1 - b) * half, half), :]
                dst = rbufs[c].at[pl.ds(RB_OFFS[c][0], half), :]
            elif ph == 1:
                half = s >> 2
                b = pb[c][1][1]
                src = acc.at[pl.ds(offs[c] + (1 - b) * half, half), :]
                dst = rbufs[c].at[pl.ds(RB_OFFS[c][1], half), :]
            elif ph == 2:
                src = acc.at[pl.ds(offs[c], s >> 2), :]
                dst = rbufs[c].at[pl.ds(RB_OFFS[c][2], s >> 2), :]
            elif ph == 3:
                src = ov.at[pl.ds(offs[c], s >> 2), :]
                dst = ov.at[pl.ds(offs[c], s >> 2), :]
            else:
                src = ov.at[pl.ds(offs[c], s >> 1), :]
                dst = ov.at[pl.ds(offs[c], s >> 1), :]
            return pltpu.make_async_remote_copy(
                src_ref=src, dst_ref=dst,
                send_sem=send_sems.at[c, ph],
                recv_sem=recv_sems.at[c, ph],
                device_id=(partner,),
                device_id_type=pl.DeviceIdType.MESH,
            )

        def out_flush(c, slot, off, size):
            cp = pltpu.make_async_copy(
                ov.at[pl.ds(off, size), :],
                out_ref.at[pl.ds(off, size), :],
                ocp_sems.at[c, slot],
            )
            cp.start()
            return cp

        offs = [jnp.int32(b) for b in BASES]
        rdmas = {}
        out_copies = []
        for c in ORDER:
            rdmas[c] = make_rdma(c, 0, offs)
            rdmas[c].start()

        for ph in range(4):
            for c in ORDER:
                s = SIZES[c]
                rdmas[c].wait()
                if ph == 0:
                    b = pb[c][0][1]
                    keep = pl.ds(offs[c] + b * (s >> 1), s >> 1)
                    x_copies[c].wait()
                    acc[keep, :] = (
                        acc[keep, :]
                        + rbufs[c][pl.ds(RB_OFFS[c][0], s >> 1), :]
                    )
                    offs[c] = offs[c] + b * (s >> 1)
                elif ph == 1:
                    b = pb[c][1][1]
                    keep = pl.ds(offs[c] + b * (s >> 2), s >> 2)
                    acc[keep, :] = (
                        acc[keep, :]
                        + rbufs[c][pl.ds(RB_OFFS[c][1], s >> 2), :]
                    )
                    offs[c] = offs[c] + b * (s >> 2)
                elif ph == 2:
                    own = pl.ds(offs[c], s >> 2)
                    res_copies[c].wait()
                    y = (
                        acc[own, :]
                        + rbufs[c][pl.ds(RB_OFFS[c][2], s >> 2), :]
                        + res_loc[pl.ds(RES_OFFS[c], s >> 2), :]
                    )
                    rms = jnp.sqrt(
                        jnp.mean(y * y, axis=-1, keepdims=True) + 1e-6
                    )
                    ov[own, :] = y / rms * gamma_ref[:, :]
                    out_copies.append(out_flush(c, 0, offs[c], s >> 2))
                else:
                    b1 = pb[c][1][1]
                    got = offs[c] - b1 * (s >> 2) + (1 - b1) * (s >> 2)
                    out_copies.append(out_flush(c, 1, got, s >> 2))
                    offs[c] = offs[c] - b1 * (s >> 2)
                rdmas[c] = make_rdma(c, ph + 1, offs)
                rdmas[c].start()

        for c in ORDER:
            s = SIZES[c]
            rdmas[c].wait()
            b0 = pb[c][0][1]
            got = offs[c] - b0 * (s >> 1) + (1 - b0) * (s >> 1)
            out_copies.append(out_flush(c, 2, got, s >> 1))
        for cp in out_copies:
            cp.wait()

    return pl.pallas_call(
        body,
        out_shape=jax.ShapeDtypeStruct((M, D), jnp.float32),
        in_specs=[
            pl.BlockSpec(memory_space=pl.ANY),
            pl.BlockSpec(memory_space=pl.ANY),
            pl.BlockSpec(memory_space=pltpu.VMEM),
        ],
        out_specs=pl.BlockSpec(memory_space=pl.ANY),
        scratch_shapes=[
            pltpu.VMEM((M, D), jnp.float32),
            pltpu.VMEM((M, D), jnp.float32),
            *[pltpu.VMEM((s, D), jnp.float32) for s in SIZES],
            pltpu.VMEM((256, D), jnp.float32),
            pltpu.SemaphoreType.DMA((6, 5)),
            pltpu.SemaphoreType.DMA((6, 5)),
            pltpu.SemaphoreType.DMA((6,)),
            pltpu.SemaphoreType.DMA((6,)),
            pltpu.SemaphoreType.DMA((6, 3)),
        ],
        compiler_params=pltpu.CompilerParams(collective_id=0),
    )(partial, resid, gamma2)


# device time: 43296 ns/iter; 1.0505x vs baseline; 1.0505x over previous
import jax
import jax.numpy as jnp
from jax import lax
from jax.experimental import pallas as pl
from jax.experimental.pallas import tpu as pltpu

N_DEV = 8
M = 1024
D = 1024

SIZES = (192, 160, 192, 160, 160, 160)
BASES = (0, 192, 352, 544, 704, 864)
ORDERS = (
    ("x", "y", "z"), ("x", "z", "y"),
    ("y", "z", "x"), ("y", "x", "z"),
    ("z", "x", "y"), ("z", "y", "x"),
)
RB_OFFS = tuple((0, s // 2, 3 * s // 4) for s in SIZES)
RES_OFFS = (0, 48, 88, 136, 176, 216)
ORDER = (1, 3, 4, 5, 0, 2)


def _partner_and_bit(p, axis):
    q = lax.rem(p, 4)
    zc = lax.div(p, 4)
    if axis == "x":
        return zc * 4 + jnp.bitwise_xor(q, 1), jnp.bitwise_and(
            jnp.bitwise_xor(q, lax.div(q, 2)), 1
        )
    if axis == "y":
        return zc * 4 + (3 - q), lax.div(q, 2)
    return jnp.bitwise_xor(p, 4), zc


def kernel(partial, resid, gamma):
    gamma2 = gamma.reshape(1, D)

    def body(x_ref, resid_ref, gamma_ref, out_ref,
             acc, ov, rb0, rb1, rb2, rb3, rb4, rb5, res_loc,
             send_sems, recv_sems, res_sems, ocp_sems):
        p = lax.axis_index("i")
        rbufs = (rb0, rb1, rb2, rb3, rb4, rb5)

        pb = [[_partner_and_bit(p, ORDERS[c][j]) for j in range(3)]
              for c in range(6)]

        barrier_sem = pltpu.get_barrier_semaphore()
        for axis in ("x", "y", "z"):
            pl.semaphore_signal(
                barrier_sem, inc=1,
                device_id=(_partner_and_bit(p, axis)[0],),
                device_id_type=pl.DeviceIdType.MESH,
            )
        pl.semaphore_wait(barrier_sem, 3)

        res_copies = []
        for c in range(6):
            s = SIZES[c]
            b0, b1 = pb[c][0][1], pb[c][1][1]
            keep0 = BASES[c] + b0 * (s >> 1)
            off_fin = keep0 + b1 * (s >> 2)
            cp = pltpu.make_async_copy(
                resid_ref.at[pl.ds(off_fin, s >> 2), :],
                res_loc.at[pl.ds(RES_OFFS[c], s >> 2), :],
                res_sems.at[c],
            )
            cp.start()
            res_copies.append(cp)

        def make_rdma(c, ph, offs):
            s = SIZES[c]
            partner = pb[c][(0, 1, 2, 1, 0)[ph]][0]
            if ph == 0:
                half = s >> 1
                b = pb[c][0][1]
                src = x_ref.at[0, pl.ds(offs[c] + (1 - b) * half, half), :]
                dst = rbufs[c].at[pl.ds(RB_OFFS[c][0], half), :]
            elif ph == 1:
                half = s >> 2
                b = pb[c][1][1]
                src = acc.at[pl.ds(offs[c] + (1 - b) * half, half), :]
                dst = rbufs[c].at[pl.ds(RB_OFFS[c][1], half), :]
            elif ph == 2:
                src = acc.at[pl.ds(offs[c], s >> 2), :]
                dst = rbufs[c].at[pl.ds(RB_OFFS[c][2], s >> 2), :]
            elif ph == 3:
                src = ov.at[pl.ds(offs[c], s >> 2), :]
                dst = ov.at[pl.ds(offs[c], s >> 2), :]
            else:
                src = ov.at[pl.ds(offs[c], s >> 1), :]
                dst = ov.at[pl.ds(offs[c], s >> 1), :]
            return pltpu.make_async_remote_copy(
                src_ref=src, dst_ref=dst,
                send_sem=send_sems.at[c, ph],
                recv_sem=recv_sems.at[c, ph],
                device_id=(partner,),
                device_id_type=pl.DeviceIdType.MESH,
            )

        def out_flush(c, slot, off, size):
            cp = pltpu.make_async_copy(
                ov.at[pl.ds(off, size), :],
                out_ref.at[pl.ds(off, size), :],
                ocp_sems.at[c, slot],
            )
            cp.start()
            return cp

        offs = [jnp.int32(b) for b in BASES]
        rdmas = {}
        out_copies = []
        for c in ORDER:
            rdmas[c] = make_rdma(c, 0, offs)
            rdmas[c].start()

        for ph in range(4):
            for c in ORDER:
                s = SIZES[c]
                rdmas[c].wait()
                if ph == 0:
                    b = pb[c][0][1]
                    keep = pl.ds(offs[c] + b * (s >> 1), s >> 1)
                    acc[keep, :] = (
                        x_ref[0, keep, :]
                        + rbufs[c][pl.ds(RB_OFFS[c][0], s >> 1), :]
                    )
                    offs[c] = offs[c] + b * (s >> 1)
                elif ph == 1:
                    b = pb[c][1][1]
                    keep = pl.ds(offs[c] + b * (s >> 2), s >> 2)
                    acc[keep, :] = (
                        acc[keep, :]
                        + rbufs[c][pl.ds(RB_OFFS[c][1], s >> 2), :]
                    )
                    offs[c] = offs[c] + b * (s >> 2)
                elif ph == 2:
                    own = pl.ds(offs[c], s >> 2)
                    res_copies[c].wait()
                    y = (
                        acc[own, :]
                        + rbufs[c][pl.ds(RB_OFFS[c][2], s >> 2), :]
                        + res_loc[pl.ds(RES_OFFS[c], s >> 2), :]
                    )
                    rms = jnp.sqrt(
                        jnp.mean(y * y, axis=-1, keepdims=True) + 1e-6
                    )
                    ov[own, :] = y / rms * gamma_ref[:, :]
                    out_copies.append(out_flush(c, 0, offs[c], s >> 2))
                else:
                    b1 = pb[c][1][1]
                    got = offs[c] - b1 * (s >> 2) + (1 - b1) * (s >> 2)
                    out_copies.append(out_flush(c, 1, got, s >> 2))
                    offs[c] = offs[c] - b1 * (s >> 2)
                rdmas[c] = make_rdma(c, ph + 1, offs)
                rdmas[c].start()

        for c in ORDER:
            s = SIZES[c]
            rdmas[c].wait()
            b0 = pb[c][0][1]
            got = offs[c] - b0 * (s >> 1) + (1 - b0) * (s >> 1)
            out_copies.append(out_flush(c, 2, got, s >> 1))
        for cp in out_copies:
            cp.wait()

    return pl.pallas_call(
        body,
        out_shape=jax.ShapeDtypeStruct((M, D), jnp.float32),
        in_specs=[
            pl.BlockSpec(memory_space=pltpu.VMEM),
            pl.BlockSpec(memory_space=pl.ANY),
            pl.BlockSpec(memory_space=pltpu.VMEM),
        ],
        out_specs=pl.BlockSpec(memory_space=pl.ANY),
        scratch_shapes=[
            pltpu.VMEM((M, D), jnp.float32),
            pltpu.VMEM((M, D), jnp.float32),
            *[pltpu.VMEM((s, D), jnp.float32) for s in SIZES],
            pltpu.VMEM((256, D), jnp.float32),
            pltpu.SemaphoreType.DMA((6, 5)),
            pltpu.SemaphoreType.DMA((6, 5)),
            pltpu.SemaphoreType.DMA((6,)),
            pltpu.SemaphoreType.DMA((6, 3)),
        ],
        compiler_params=pltpu.CompilerParams(collective_id=0),
    )(partial, resid, gamma2)


# device time: 42733 ns/iter; 1.0643x vs baseline; 1.0132x over previous
import jax
import jax.numpy as jnp
from jax import lax
from jax.experimental import pallas as pl
from jax.experimental.pallas import tpu as pltpu

N_DEV = 8
M = 1024
D = 1024

SIZES = (192, 160, 192, 160, 160, 160)
BASES = (0, 192, 352, 544, 704, 864)
ORDERS = (
    ("x", "y", "z"), ("x", "z", "y"),
    ("y", "z", "x"), ("y", "x", "z"),
    ("z", "x", "y"), ("z", "y", "x"),
)
RB_OFFS = tuple((0, s // 4, s // 2, 3 * s // 4) for s in SIZES)
RES_OFFS = (0, 48, 88, 136, 176, 216)
ORDER = (1, 3, 4, 5, 0, 2)
P0A, P0B, P1, P2, P3, P4A, P4B = range(7)


def _partner_and_bit(p, axis):
    q = lax.rem(p, 4)
    zc = lax.div(p, 4)
    if axis == "x":
        return zc * 4 + jnp.bitwise_xor(q, 1), jnp.bitwise_and(
            jnp.bitwise_xor(q, lax.div(q, 2)), 1
        )
    if axis == "y":
        return zc * 4 + (3 - q), lax.div(q, 2)
    return jnp.bitwise_xor(p, 4), zc


def kernel(partial, resid, gamma):
    gamma2 = gamma.reshape(1, D)

    def body(x_ref, resid_ref, gamma_ref, out_ref,
             acc, ov, rb0, rb1, rb2, rb3, rb4, rb5, res_loc,
             send_sems, recv_sems, res_sems, ocp_sems):
        p = lax.axis_index("i")
        rbufs = (rb0, rb1, rb2, rb3, rb4, rb5)

        pb = [[_partner_and_bit(p, ORDERS[c][j]) for j in range(3)]
              for c in range(6)]

        barrier_sem = pltpu.get_barrier_semaphore()
        for axis in ("x", "y", "z"):
            pl.semaphore_signal(
                barrier_sem, inc=1,
                device_id=(_partner_and_bit(p, axis)[0],),
                device_id_type=pl.DeviceIdType.MESH,
            )
        pl.semaphore_wait(barrier_sem, 3)

        q4 = [s >> 2 for s in SIZES]
        keep0, send0, subA, subB, own, got3, in4a, in4b = (
            [None] * 6, [None] * 6, [None] * 6, [None] * 6,
            [None] * 6, [None] * 6, [None] * 6, [None] * 6,
        )
        for c in range(6):
            s = SIZES[c]
            b0, b1 = pb[c][0][1], pb[c][1][1]
            keep0[c] = BASES[c] + b0 * (s >> 1)
            send0[c] = BASES[c] + (1 - b0) * (s >> 1)
            subA[c] = keep0[c] + (1 - b1) * q4[c]
            subB[c] = keep0[c] + b1 * q4[c]
            own[c] = subB[c]
            got3[c] = subA[c]
            in4a[c] = send0[c] + b1 * q4[c]
            in4b[c] = send0[c] + (1 - b1) * q4[c]
        sendA = [send0[c] + (1 - pb[c][1][1]) * q4[c] for c in range(6)]
        sendB = [send0[c] + pb[c][1][1] * q4[c] for c in range(6)]

        res_copies = []
        for c in range(6):
            cp = pltpu.make_async_copy(
                resid_ref.at[pl.ds(own[c], q4[c]), :],
                res_loc.at[pl.ds(RES_OFFS[c], q4[c]), :],
                res_sems.at[c],
            )
            cp.start()
            res_copies.append(cp)

        def ex(c, ph, src, dst, axis_pos):
            return pltpu.make_async_remote_copy(
                src_ref=src, dst_ref=dst,
                send_sem=send_sems.at[c, ph],
                recv_sem=recv_sems.at[c, ph],
                device_id=(pb[c][axis_pos][0],),
                device_id_type=pl.DeviceIdType.MESH,
            )

        def out_flush(c, slot, off, size):
            cp = pltpu.make_async_copy(
                ov.at[pl.ds(off, size), :],
                out_ref.at[pl.ds(off, size), :],
                ocp_sems.at[c, slot],
            )
            cp.start()
            return cp

        r = {}
        out_copies = []
        for c in ORDER:
            r[c, P0A] = ex(c, P0A, x_ref.at[0, pl.ds(sendA[c], q4[c]), :],
                           rbufs[c].at[pl.ds(RB_OFFS[c][0], q4[c]), :], 0)
            r[c, P0A].start()
        for c in ORDER:
            r[c, P0B] = ex(c, P0B, x_ref.at[0, pl.ds(sendB[c], q4[c]), :],
                           rbufs[c].at[pl.ds(RB_OFFS[c][1], q4[c]), :], 0)
            r[c, P0B].start()
        for c in ORDER:
            n = q4[c]
            r[c, P0A].wait()
            a = pl.ds(subA[c], n)
            acc[a, :] = x_ref[0, a, :] + rbufs[c][pl.ds(RB_OFFS[c][0], n), :]
            r[c, P1] = ex(c, P1, acc.at[a, :],
                          rbufs[c].at[pl.ds(RB_OFFS[c][2], n), :], 1)
            r[c, P1].start()
        for c in ORDER:
            n = q4[c]
            r[c, P0B].wait()
            bq = pl.ds(subB[c], n)
            acc[bq, :] = x_ref[0, bq, :] + rbufs[c][pl.ds(RB_OFFS[c][1], n), :]
        for c in ORDER:
            n = q4[c]
            r[c, P1].wait()
            bq = pl.ds(own[c], n)
            acc[bq, :] = acc[bq, :] + rbufs[c][pl.ds(RB_OFFS[c][2], n), :]
            r[c, P2] = ex(c, P2, acc.at[bq, :],
                          rbufs[c].at[pl.ds(RB_OFFS[c][3], n), :], 2)
            r[c, P2].start()
        for c in ORDER:
            n = q4[c]
            r[c, P2].wait()
            bq = pl.ds(own[c], n)
            res_copies[c].wait()
            y = (
                acc[bq, :]
                + rbufs[c][pl.ds(RB_OFFS[c][3], n), :]
                + res_loc[pl.ds(RES_OFFS[c], n), :]
            )
            rms = jnp.sqrt(jnp.mean(y * y, axis=-1, keepdims=True) + 1e-6)
            ov[bq, :] = y / rms * gamma_ref[:, :]
            out_copies.append(out_flush(c, 0, own[c], n))
            r[c, P3] = ex(c, P3, ov.at[bq, :], ov.at[bq, :], 1)
            r[c, P3].start()
            r[c, P4A] = ex(c, P4A, ov.at[bq, :], ov.at[bq, :], 0)
            r[c, P4A].start()
        for c in ORDER:
            n = q4[c]
            r[c, P3].wait()
            g = pl.ds(got3[c], n)
            out_copies.append(out_flush(c, 1, got3[c], n))
            r[c, P4B] = ex(c, P4B, ov.at[g, :], ov.at[g, :], 0)
            r[c, P4B].start()
        for c in ORDER:
            r[c, P4A].wait()
            out_copies.append(out_flush(c, 2, in4a[c], q4[c]))
        for c in ORDER:
            r[c, P4B].wait()
            out_copies.append(out_flush(c, 3, in4b[c], q4[c]))
        for cp in out_copies:
            cp.wait()

    return pl.pallas_call(
        body,
        out_shape=jax.ShapeDtypeStruct((M, D), jnp.float32),
        in_specs=[
            pl.BlockSpec(memory_space=pltpu.VMEM),
            pl.BlockSpec(memory_space=pl.ANY),
            pl.BlockSpec(memory_space=pltpu.VMEM),
        ],
        out_specs=pl.BlockSpec(memory_space=pl.ANY),
        scratch_shapes=[
            pltpu.VMEM((M, D), jnp.float32),
            pltpu.VMEM((M, D), jnp.float32),
            *[pltpu.VMEM((s, D), jnp.float32) for s in SIZES],
            pltpu.VMEM((256, D), jnp.float32),
            pltpu.SemaphoreType.DMA((6, 7)),
            pltpu.SemaphoreType.DMA((6, 7)),
            pltpu.SemaphoreType.DMA((6,)),
            pltpu.SemaphoreType.DMA((6, 4)),
        ],
        compiler_params=pltpu.CompilerParams(collective_id=0),
    )(partial, resid, gamma2)
